# baseline (speedup 1.0000x reference)
import functools

import jax
import jax.numpy as jnp
from jax import lax
from jax.experimental import pallas as pl
from jax.experimental.pallas import tpu as pltpu
from jax.experimental.pallas import tpu_sc as plsc

B = 4096
L = 200
DIM = 64
PAD = 128
VOCAB = 100000
ROWS = B * L

NC = 2
NS = 16
NW = NC * NS
ROWS_PER_W = ROWS // NW
CHUNK = 400
N_CHUNKS = ROWS_PER_W // CHUNK

_sc_mesh = plsc.VectorSubcoreMesh(core_axis_name="c", subcore_axis_name="s")


@functools.partial(
    pl.kernel,
    mesh=_sc_mesh,
    out_type=jax.ShapeDtypeStruct((ROWS, PAD), jnp.float32),
    scratch_types=[
        pltpu.VMEM((ROWS_PER_W,), jnp.int32),
        pltpu.VMEM((CHUNK, PAD), jnp.float32),
        pltpu.VMEM((CHUNK, PAD), jnp.float32),
        pltpu.SemaphoreType.DMA,
        pltpu.SemaphoreType.DMA,
        pltpu.SemaphoreType.DMA,
        pltpu.SemaphoreType.DMA,
    ],
)
def _sc_gather(idx_hbm, table_hbm, out_hbm, idx_v, rows0, rows1, sg0, sg1, so0, so1):
    wid = lax.axis_index("s") * NC + lax.axis_index("c")
    base = wid * ROWS_PER_W
    pltpu.sync_copy(idx_hbm.at[pl.ds(base, ROWS_PER_W)], idx_v)
    rows = (rows0, rows1)
    sg = (sg0, sg1)
    so = (so0, so1)

    def start_gather(i, b):
        pltpu.async_copy(
            table_hbm.at[idx_v.at[pl.ds(i * CHUNK, CHUNK)]], rows[b], sg[b]
        )

    def wait_gather(b):
        pltpu.make_async_copy(
            table_hbm.at[idx_v.at[pl.ds(0, CHUNK)]], rows[b], sg[b]
        ).wait()

    def start_out(i, b):
        pltpu.async_copy(rows[b], out_hbm.at[pl.ds(base + i * CHUNK, CHUNK)], so[b])

    def wait_out(b):
        pltpu.make_async_copy(
            rows[b], out_hbm.at[pl.ds(base, CHUNK)], so[b]
        ).wait()

    def body(t, carry):
        for bb in (0, 1):
            i = 2 * t + bb
            if bb == 0:
                @pl.when(t >= 1)
                def _():
                    wait_out(0)
            else:
                @pl.when(t >= 1)
                def _():
                    wait_out(1)
            start_gather(i, bb)
            if bb == 0:
                @pl.when(t >= 1)
                def _():
                    wait_gather(1)
                    start_out(i - 1, 1)
            else:
                wait_gather(0)
                start_out(i - 1, 0)
        return carry

    lax.fori_loop(0, N_CHUNKS // 2, body, 0)
    wait_gather(1)
    start_out(N_CHUNKS - 1, 1)
    wait_out(0)
    wait_out(1)


DBLK = 4096


def _dense_body(e_ref, wc_ref, bc_ref, wa_ref, ba_ref, o_ref):
    e = e_ref[:, :DIM]
    h = jnp.tanh(
        jax.lax.dot(e, wc_ref[...], preferred_element_type=jnp.float32)
        + bc_ref[...]
    )
    o_ref[...] = (
        e
        + jax.lax.dot(h, wa_ref[...], preferred_element_type=jnp.float32)
        + ba_ref[...]
    )


def _dense(emb, wc, bc, wa, ba):
    return pl.pallas_call(
        _dense_body,
        grid=(ROWS // DBLK,),
        in_specs=[
            pl.BlockSpec((DBLK, PAD), lambda i: (i, 0)),
            pl.BlockSpec((DIM, 3 * DIM), lambda i: (0, 0)),
            pl.BlockSpec((1, 3 * DIM), lambda i: (0, 0)),
            pl.BlockSpec((3 * DIM, DIM), lambda i: (0, 0)),
            pl.BlockSpec((1, DIM), lambda i: (0, 0)),
        ],
        out_specs=pl.BlockSpec((DBLK, DIM), lambda i: (i, 0)),
        out_shape=jax.ShapeDtypeStruct((ROWS, DIM), jnp.float32),
    )(emb, wc, bc, wa, ba)


def kernel(phoneme_input, table, W_ph, b_ph, W_em, b_em, W_vo, b_vo, W_al, b_al):
    idx = phoneme_input.reshape(-1).astype(jnp.int32)
    tpad = jnp.pad(table, ((0, 0), (0, PAD - DIM)))
    emb = _sc_gather(idx, tpad)
    wc = jnp.concatenate([W_ph.T, W_em.T, W_vo.T], axis=1)
    bc = jnp.concatenate([b_ph, b_em, b_vo]).reshape(1, 3 * DIM)
    wa = W_al.T
    ba = b_al.reshape(1, DIM)
    return _dense(emb, wc, bc, wa, ba).reshape(B, L, DIM)
